# baseline (speedup 1.0000x reference)
import jax
import jax.numpy as jnp
from jax import lax
from jax.experimental import pallas as pl
from jax.experimental.pallas import tpu as pltpu
from jax.experimental.pallas import tpu_sc as plsc

_FEATURE_DIM = 1000000
_EMBED_DIM = 16
_BATCH = 4096
_N_FIELDS = 26

_NC = 2
_NS = 16
_NW = _NC * _NS
_BPW = _BATCH // _NW
_KPW = _BPW * _N_FIELDS
_RPC = 4
_CHUNK = _RPC * _N_FIELDS
_NCHUNK = _BPW // _RPC
_NBUF = 4
_PADF = 2 * _EMBED_DIM


def _bcast_lane(vec, lane):
    return lax.gather(
        vec,
        jnp.full((_EMBED_DIM, 1), lane, jnp.int32),
        lax.GatherDimensionNumbers(
            offset_dims=(),
            collapsed_slice_dims=(0,),
            start_index_map=(0,),
        ),
        slice_sizes=(1,),
        mode=lax.GatherScatterMode.PROMISE_IN_BOUNDS,
    )


def _fm_body(table, idxs, vals, boffs, out, idx_v, vals_v, boff_v, out_v,
             *bufs_and_sems):
    bufs = bufs_and_sems[:_NBUF]
    sems = bufs_and_sems[_NBUF:]
    wid = lax.axis_index("s") * _NC + lax.axis_index("c")

    pltpu.sync_copy(idxs.at[wid], idx_v)
    pltpu.sync_copy(vals.at[wid], vals_v)
    pltpu.sync_copy(boffs.at[wid], boff_v)

    iota = lax.iota(jnp.int32, _EMBED_DIM)
    for j in range(_NBUF):
        pltpu.async_copy(table.at[idx_v.at[j]], bufs[j], sems[j])

    @pl.loop(0, _NCHUNK, step=_NBUF)
    def _outer(jbase):
        for slot in range(_NBUF):
            j = jbase + slot
            buf = bufs[slot]
            pltpu.make_async_copy(
                table.at[idx_v.at[0]], buf, sems[slot]
            ).wait()

            def rbody(i, carry, _buf=buf, _j=j):
                b = _j * _RPC + i
                v0 = vals_v[b, pl.ds(0, _EMBED_DIM)]
                v1 = vals_v[b, pl.ds(_EMBED_DIM, _EMBED_DIM)]
                o0 = boff_v[b, pl.ds(0, _EMBED_DIM)]
                o1 = boff_v[b, pl.ds(_EMBED_DIM, _EMBED_DIM)]
                s = jnp.zeros((_EMBED_DIM,), jnp.float32)
                q = jnp.zeros((_EMBED_DIM,), jnp.float32)
                for f in range(_N_FIELDS):
                    lane = f % _EMBED_DIM
                    pos = i * _N_FIELDS + f
                    col = (
                        _bcast_lane(o0 if f < _EMBED_DIM else o1, lane) + iota
                    )
                    row = jnp.full((_EMBED_DIM,), pos, jnp.int32)
                    e = plsc.load_gather(_buf, [row, col])
                    w = _bcast_lane(v0 if f < _EMBED_DIM else v1, lane)
                    t = w * e
                    s = s + t
                    q = q + t * t
                out_v[b, :] = 0.5 * (s * s - q)
                return carry

            lax.fori_loop(0, _RPC, rbody, 0)

            @pl.when(j + _NBUF < _NCHUNK)
            def _refire(_buf=buf, _slot=slot, _j=j):
                pltpu.async_copy(
                    table.at[idx_v.at[_j + _NBUF]], _buf, sems[_slot]
                )

    pltpu.sync_copy(out_v, out.at[pl.ds(wid * _BPW, _BPW), :])


@jax.jit
def kernel(feature_embedding, feature_idx, feature_vals):
    table2 = feature_embedding.reshape(_FEATURE_DIM // 8, 8 * _EMBED_DIM)
    blk = (feature_idx >> 3).reshape(_NW, _NCHUNK, _CHUNK)
    boff = jnp.pad(
        (feature_idx & 7) * _EMBED_DIM,
        ((0, 0), (0, _PADF - _N_FIELDS)),
    ).reshape(_NW, _BPW, _PADF)
    vals_r = jnp.pad(
        feature_vals, ((0, 0), (0, _PADF - _N_FIELDS))
    ).reshape(_NW, _BPW, _PADF)

    mesh = plsc.VectorSubcoreMesh(
        core_axis_name="c", subcore_axis_name="s",
        num_cores=_NC, num_subcores=_NS,
    )
    run = pl.kernel(
        _fm_body,
        out_type=jax.ShapeDtypeStruct((_BATCH, _EMBED_DIM), jnp.float32),
        mesh=mesh,
        scratch_types=[
            pltpu.VMEM((_NCHUNK, _CHUNK), jnp.int32),
            pltpu.VMEM((_BPW, _PADF), jnp.float32),
            pltpu.VMEM((_BPW, _PADF), jnp.int32),
            pltpu.VMEM((_BPW, _EMBED_DIM), jnp.float32),
        ]
        + [pltpu.VMEM((_CHUNK, 8 * _EMBED_DIM), jnp.float32)] * _NBUF
        + [pltpu.SemaphoreType.DMA] * _NBUF,
        compiler_params=pltpu.CompilerParams(needs_layout_passes=False),
    )
    return run(table2, blk, vals_r, boff)
